# baseline (speedup 1.0000x reference)
import functools

import jax
import jax.numpy as jnp
from jax import lax
from jax.experimental import pallas as pl
from jax.experimental.pallas import tpu as pltpu
from jax.experimental.pallas import tpu_sc as plsc

_L = 16


def _rsqrt_newton(x):
    i = lax.bitcast_convert_type(x, jnp.int32)
    y = lax.bitcast_convert_type(jnp.int32(0x5F3759DF) - (i >> 1), jnp.float32)
    for _ in range(3):
        y = y * (jnp.float32(1.5) - jnp.float32(0.5) * x * y * y)
    return jnp.where(x > jnp.float32(0.0), y, jnp.float32(0.0))


def _make_sc_kernel(B, C, H, W, P, NC, NS):
    HW = H * W
    NW = NC * NS
    BP = B * P
    RB = 128
    n_rb = BP // RB
    n_cg = NW // n_rb
    c_w = C // n_cg
    n_ct = C // 8
    ct_w = c_w // 8
    rb_sc = n_rb // NC
    nrv = RB // _L
    ntc = W // 128
    mesh = plsc.VectorSubcoreMesh(core_axis_name="c", subcore_axis_name="s")

    @functools.partial(
        pl.kernel,
        mesh=mesh,
        out_type=(
            jax.ShapeDtypeStruct((BP * C,), jnp.float32),
            jax.ShapeDtypeStruct((P,), jnp.int32),
        ),
        compiler_params=pltpu.CompilerParams(needs_layout_passes=False),
        scratch_types=[
            pltpu.VMEM((P,), jnp.int32),
            pltpu.VMEM((c_w * RB,), jnp.int32),
            pltpu.VMEM((c_w * RB,), jnp.float32),
            pltpu.VMEM((ct_w * 8 * RB,), jnp.float32),
            pltpu.VMEM((RB,), jnp.float32),
            pltpu.VMEM((n_cg * RB,), jnp.float32),
            pltpu.VMEM_SHARED((rb_sc * n_cg * RB,), jnp.float32),
            pltpu.SemaphoreType.DMA,
        ],
    )
    def k(feat_hbm, pids_hbm, out_hbm, pid_out_hbm,
          pids_v, idx_v, vals_v, out_v, part_v, four_v, shared, sem):
        s = lax.axis_index("s")
        c_ax = lax.axis_index("c")
        rb_l = s % rb_sc
        rb = c_ax * rb_sc + rb_l
        cg = s // rb_sc
        c0 = cg * c_w

        pltpu.sync_copy(pids_hbm, pids_v)

        @pl.when(jnp.logical_and(s == 0, c_ax == 0))
        def _():
            pltpu.sync_copy(pids_v, pid_out_hbm)

        b = (rb * RB) // P
        pstart = (rb * RB) % P
        pv = []
        for k16 in range(nrv):
            pid = pids_v[pl.ds(pstart + k16 * _L, _L)]
            h = pid // W
            w = pid - h * W
            off = (((h >> 3) * ntc + (w >> 7)) << 10) + ((h & 7) << 7) + (w & 127)
            pv.append(off)

        base = (b * C + c0) * HW

        def build_fire(cc, carry):
            rowb = base + cc * HW
            for k16 in range(nrv):
                idx_v[pl.ds(cc * RB + k16 * _L, _L)] = pv[k16] + rowb
            pltpu.async_copy(
                feat_hbm.at[idx_v.at[pl.ds(cc * RB, RB)]],
                vals_v.at[pl.ds(cc * RB, RB)],
                sem,
            )
            return carry

        lax.fori_loop(0, c_w, build_fire, 0, unroll=2)
        pltpu.make_async_copy(feat_hbm.at[pl.ds(0, c_w * RB)], vals_v, sem).wait()

        def sumsq(cc, acc):
            out = []
            for k16 in range(nrv):
                v = vals_v[pl.ds(cc * RB + k16 * _L, _L)]
                out.append(acc[k16] + v * v)
            return tuple(out)

        acc = lax.fori_loop(
            0, c_w, sumsq,
            tuple(jnp.zeros((_L,), jnp.float32) for _ in range(nrv)),
            unroll=4,
        )

        for k16 in range(nrv):
            part_v[pl.ds(k16 * _L, _L)] = acc[k16]
        pltpu.sync_copy(part_v, shared.at[pl.ds((rb_l * n_cg + cg) * RB, RB)])
        plsc.subcore_barrier()
        pltpu.sync_copy(shared.at[pl.ds(rb_l * n_cg * RB, n_cg * RB)], four_v)

        den = []
        for k16 in range(nrv):
            tot = jnp.zeros((_L,), jnp.float32)
            for g in range(n_cg):
                tot = tot + four_v[pl.ds(g * RB + k16 * _L, _L)]
            r = _rsqrt_newton(tot)
            den.append(jnp.float32(1.0) / (tot * r + jnp.float32(1e-7)))

        def norm(cc, carry):
            for k16 in range(nrv):
                v = vals_v[pl.ds(cc * RB + k16 * _L, _L)]
                out_v[pl.ds((cc // 8) * (8 * RB) + (cc % 8) * RB + k16 * _L, _L)] = v * den[k16]
            return carry

        lax.fori_loop(0, c_w, norm, 0, unroll=2)

        for t in range(ct_w):
            ct = cg * ct_w + t
            pltpu.sync_copy(
                out_v.at[pl.ds(t * (8 * RB), 8 * RB)],
                out_hbm.at[pl.ds((ct * n_rb + rb) * (8 * RB), 8 * RB)],
            )

    return k


@functools.lru_cache(maxsize=None)
def _build(B, C, H, W, P):
    info = plsc.get_sparse_core_info()
    return _make_sc_kernel(B, C, H, W, P, info.num_cores, info.num_subcores)


def kernel(feat, num_patches, patch_ids):
    B, C, H, W = feat.shape
    P = patch_ids.shape[0]
    feat_flat = (
        feat.reshape(B, C, H // 8, 8, W // 128, 128)
        .transpose(0, 1, 2, 4, 3, 5)
        .reshape(B * C * H * W)
    )
    pids = patch_ids.astype(jnp.int32)
    out_flat, pid_out = _build(B, C, H, W, P)(feat_flat, pids)
    out = (
        out_flat.reshape(C // 8, B * P // 128, 8, 128)
        .transpose(1, 3, 0, 2)
        .reshape(B * P, C)
    )
    return out, pid_out


# speedup vs baseline: 1.0365x; 1.0365x over previous
import functools

import jax
import jax.numpy as jnp
from jax import lax
from jax.experimental import pallas as pl
from jax.experimental.pallas import tpu as pltpu
from jax.experimental.pallas import tpu_sc as plsc

_L = 16


def _rsqrt_newton(x):
    i = lax.bitcast_convert_type(x, jnp.int32)
    y = lax.bitcast_convert_type(jnp.int32(0x5F3759DF) - (i >> 1), jnp.float32)
    for _ in range(3):
        y = y * (jnp.float32(1.5) - jnp.float32(0.5) * x * y * y)
    return jnp.where(x > jnp.float32(0.0), y, jnp.float32(0.0))


def _make_sc_kernel(B, C, H, W, P, NC, NS):
    HW = H * W
    NW = NC * NS
    BP = B * P
    RB = 128
    n_rb = BP // RB
    n_cg = NW // n_rb
    c_w = C // n_cg
    n_ct = C // 8
    ct_w = c_w // 8
    rb_sc = n_rb // NC
    nrv = RB // _L
    ntc = W // 128
    mesh = plsc.VectorSubcoreMesh(core_axis_name="c", subcore_axis_name="s")

    @functools.partial(
        pl.kernel,
        mesh=mesh,
        out_type=(
            jax.ShapeDtypeStruct((BP * C,), jnp.float32),
            jax.ShapeDtypeStruct((P,), jnp.int32),
        ),
        compiler_params=pltpu.CompilerParams(
            needs_layout_passes=False, skip_device_barrier=True
        ),
        scratch_types=[
            pltpu.VMEM((P,), jnp.int32),
            pltpu.VMEM((c_w * RB,), jnp.int32),
            pltpu.VMEM((c_w * RB,), jnp.float32),
            pltpu.VMEM((ct_w * 8 * RB,), jnp.float32),
            pltpu.VMEM((RB,), jnp.float32),
            pltpu.VMEM((n_cg * RB,), jnp.float32),
            pltpu.VMEM_SHARED((rb_sc * n_cg * RB,), jnp.float32),
            pltpu.SemaphoreType.DMA,
        ],
    )
    def k(feat_hbm, pids_hbm, out_hbm, pid_out_hbm,
          pids_v, idx_v, vals_v, out_v, part_v, four_v, shared, sem):
        s = lax.axis_index("s")
        c_ax = lax.axis_index("c")
        rb_l = s % rb_sc
        rb = c_ax * rb_sc + rb_l
        cg = s // rb_sc
        c0 = cg * c_w

        pltpu.sync_copy(pids_hbm, pids_v)

        @pl.when(jnp.logical_and(s == 0, c_ax == 0))
        def _():
            pltpu.sync_copy(pids_v, pid_out_hbm)

        b = (rb * RB) // P
        pstart = (rb * RB) % P
        pv = []
        for k16 in range(nrv):
            pid = pids_v[pl.ds(pstart + k16 * _L, _L)]
            h = pid // W
            w = pid - h * W
            off = (((h >> 3) * ntc + (w >> 7)) << 10) + ((h & 7) << 7) + (w & 127)
            pv.append(off)

        base = (b * C + c0) * HW

        def build_fire(cc, carry):
            rowb = base + cc * HW
            for k16 in range(nrv):
                idx_v[pl.ds(cc * RB + k16 * _L, _L)] = pv[k16] + rowb
            pltpu.async_copy(
                feat_hbm.at[idx_v.at[pl.ds(cc * RB, RB)]],
                vals_v.at[pl.ds(cc * RB, RB)],
                sem,
            )
            return carry

        lax.fori_loop(0, c_w, build_fire, 0, unroll=2)
        pltpu.make_async_copy(feat_hbm.at[pl.ds(0, c_w * RB)], vals_v, sem).wait()

        def sumsq(cc, acc):
            out = []
            for k16 in range(nrv):
                v = vals_v[pl.ds(cc * RB + k16 * _L, _L)]
                out.append(acc[k16] + v * v)
            return tuple(out)

        acc = lax.fori_loop(
            0, c_w, sumsq,
            tuple(jnp.zeros((_L,), jnp.float32) for _ in range(nrv)),
            unroll=4,
        )

        for k16 in range(nrv):
            part_v[pl.ds(k16 * _L, _L)] = acc[k16]
        pltpu.sync_copy(part_v, shared.at[pl.ds((rb_l * n_cg + cg) * RB, RB)])
        plsc.subcore_barrier()
        pltpu.sync_copy(shared.at[pl.ds(rb_l * n_cg * RB, n_cg * RB)], four_v)

        den = []
        for k16 in range(nrv):
            tot = jnp.zeros((_L,), jnp.float32)
            for g in range(n_cg):
                tot = tot + four_v[pl.ds(g * RB + k16 * _L, _L)]
            r = _rsqrt_newton(tot)
            den.append(jnp.float32(1.0) / (tot * r + jnp.float32(1e-7)))

        def norm(cc, carry):
            for k16 in range(nrv):
                v = vals_v[pl.ds(cc * RB + k16 * _L, _L)]
                out_v[pl.ds((cc // 8) * (8 * RB) + (cc % 8) * RB + k16 * _L, _L)] = v * den[k16]
            return carry

        lax.fori_loop(0, c_w, norm, 0, unroll=2)

        for t in range(ct_w):
            ct = cg * ct_w + t
            pltpu.sync_copy(
                out_v.at[pl.ds(t * (8 * RB), 8 * RB)],
                out_hbm.at[pl.ds((ct * n_rb + rb) * (8 * RB), 8 * RB)],
            )

    return k


@functools.lru_cache(maxsize=None)
def _build(B, C, H, W, P):
    info = plsc.get_sparse_core_info()
    return _make_sc_kernel(B, C, H, W, P, info.num_cores, info.num_subcores)


def kernel(feat, num_patches, patch_ids):
    B, C, H, W = feat.shape
    P = patch_ids.shape[0]
    feat_flat = (
        feat.reshape(B, C, H // 8, 8, W // 128, 128)
        .transpose(0, 1, 2, 4, 3, 5)
        .reshape(B * C * H * W)
    )
    pids = patch_ids.astype(jnp.int32)
    out_flat, pid_out = _build(B, C, H, W, P)(feat_flat, pids)
    out = (
        out_flat.reshape(C // 8, B * P // 128, 8, 128)
        .transpose(1, 3, 0, 2)
        .reshape(B * P, C)
    )
    return out, pid_out
